# baseline (speedup 1.0000x reference)
import functools

import jax
import jax.numpy as jnp
from jax import lax
from jax.experimental import pallas as pl
from jax.experimental.pallas import tpu as pltpu
from jax.experimental.pallas import tpu_sc as plsc

NTOKEN = 1000000
NINP = 32
BATCH = 16384
SEQ = 50

NC = 2
NS = 16
NW = NC * NS
BW = BATCH // NW
NBT = BW // 128
NG = NINP // 8


def _emb_body(idx_hbm, table_hbm, out_hbm, idx_v, a0, a1, b0, b1, gsems, wsems):
    wid = lax.axis_index("s") * NC + lax.axis_index("c")
    w0 = wid * NBT
    pltpu.sync_copy(idx_hbm.at[:, pl.ds(wid * BW, BW)], idx_v)

    A = (a0, a1)
    B = (b0, b1)

    def fire_gathers(l, p):
        for btl in range(NBT):
            src = table_hbm.at[idx_v.at[l, pl.ds(btl * 128, 128)]]
            pltpu.make_async_copy(src, A[p].at[pl.ds(btl * 128, 128)], gsems[p]).start()

    def wait_gathers(l, p):
        for btl in range(NBT):
            src = table_hbm.at[idx_v.at[l, pl.ds(btl * 128, 128)]]
            pltpu.make_async_copy(src, A[p].at[pl.ds(btl * 128, 128)], gsems[p]).wait()

    def fire_wb(l, p):
        for g in range(NG):
            pltpu.make_async_copy(B[p].at[g], out_hbm.at[l, g, pl.ds(w0, NBT)], wsems[p]).start()

    def wait_wb(l, p):
        for g in range(NG):
            pltpu.make_async_copy(B[p].at[g], out_hbm.at[l, g, pl.ds(w0, NBT)], wsems[p]).wait()

    def transpose(p):
        @plsc.parallel_loop(0, NINP, unroll=8)
        def _(f):
            g = f // 8
            fi = f - g * 8
            fvec = jnp.full((16,), f, jnp.int32)
            for btl in range(NBT):
                for k in range(8):
                    bvec = jnp.arange(16, dtype=jnp.int32) + (btl * 128 + k * 16)
                    v = plsc.load_gather(A[p], [bvec, fvec])
                    B[p][g, btl, fi, pl.ds(k * 16, 16)] = v

    fire_gathers(0, 0)

    def group(i, carry):
        l0 = 2 * i
        l1 = 2 * i + 1

        @pl.when(i > 0)
        def _():
            wait_wb(l1 - 2, 1)

        fire_gathers(l1, 1)
        wait_gathers(l0, 0)

        @pl.when(i > 0)
        def _():
            wait_wb(l0 - 2, 0)

        transpose(0)
        fire_wb(l0, 0)

        @pl.when(i < (SEQ // 2) - 1)
        def _():
            fire_gathers(l0 + 2, 0)

        wait_gathers(l1, 1)
        transpose(1)
        fire_wb(l1, 1)
        return carry

    lax.fori_loop(0, SEQ // 2, group, 0)
    wait_wb(SEQ - 2, 0)
    wait_wb(SEQ - 1, 1)


@jax.jit
def _emb(idxt, table):
    mesh = plsc.VectorSubcoreMesh(core_axis_name="c", subcore_axis_name="s")
    k = pl.kernel(
        _emb_body,
        mesh=mesh,
        compiler_params=pltpu.CompilerParams(
            use_tc_tiling_on_sc=False, needs_layout_passes=False
        ),
        out_type=jax.ShapeDtypeStruct((SEQ, NG, BATCH // 128, 8, 128), jnp.float32),
        scratch_types=[
            pltpu.VMEM((SEQ, BW), jnp.int32),
            pltpu.VMEM((BW, NINP), jnp.float32),
            pltpu.VMEM((BW, NINP), jnp.float32),
            pltpu.VMEM((NG, NBT, 8, 128), jnp.float32),
            pltpu.VMEM((NG, NBT, 8, 128), jnp.float32),
            [pltpu.SemaphoreType.DMA] * 2,
            [pltpu.SemaphoreType.DMA] * 2,
        ],
    )
    return k(idxt, table)


def kernel(input_ids, embed_weight):
    idxt = input_ids.T.astype(jnp.int32)
    o5 = _emb(idxt, embed_weight)
    return o5.transpose(2, 4, 0, 1, 3).reshape(BATCH, SEQ, NINP)


# speedup vs baseline: 1.0898x; 1.0898x over previous
import functools

import jax
import jax.numpy as jnp
from jax import lax
from jax.experimental import pallas as pl
from jax.experimental.pallas import tpu as pltpu
from jax.experimental.pallas import tpu_sc as plsc

NTOKEN = 1000000
NINP = 32
BATCH = 16384
SEQ = 50

NC = 2
NS = 16
NW = NC * NS
BW = BATCH // NW
NBT = BW // 128
NG = NINP // 8
NCHUNK = SEQ * NBT
NSLOT = 8


def _emb_body(idx_hbm, table_hbm, out_hbm, idx_v, *bufs_and_sems):
    a = bufs_and_sems[:NSLOT]
    b = bufs_and_sems[NSLOT:2 * NSLOT]
    gsems = bufs_and_sems[2 * NSLOT]
    wsems = bufs_and_sems[2 * NSLOT + 1]
    wid = lax.axis_index("s") * NC + lax.axis_index("c")
    w0 = wid * NBT
    pltpu.sync_copy(idx_hbm.at[:, pl.ds(wid * BW, BW)], idx_v)

    def lbt(c):
        l = c // NBT
        return l, c - l * NBT

    def fire_gather(c, s):
        l, btl = lbt(c)
        src = table_hbm.at[idx_v.at[l, pl.ds(btl * 128, 128)]]
        pltpu.make_async_copy(src, a[s], gsems[s]).start()

    def wait_gather(c, s):
        l, btl = lbt(c)
        src = table_hbm.at[idx_v.at[l, pl.ds(btl * 128, 128)]]
        pltpu.make_async_copy(src, a[s], gsems[s]).wait()

    def fire_wb(c, s):
        l, btl = lbt(c)
        for g in range(NG):
            pltpu.make_async_copy(b[s].at[g], out_hbm.at[l, g, w0 + btl], wsems[s]).start()

    def wait_wb(c, s):
        l, btl = lbt(c)
        for g in range(NG):
            pltpu.make_async_copy(b[s].at[g], out_hbm.at[l, g, w0 + btl], wsems[s]).wait()

    def transpose(s):
        @plsc.parallel_loop(0, NINP, unroll=4)
        def _(f):
            g = f // 8
            fi = f - g * 8
            fvec = jnp.full((16,), f, jnp.int32)
            for k in range(8):
                bvec = jnp.arange(16, dtype=jnp.int32) + (k * 16)
                v = plsc.load_gather(a[s], [bvec, fvec])
                b[s][g, fi, pl.ds(k * 16, 16)] = v

    for s in range(NSLOT):
        fire_gather(s, s)

    ngroup = NCHUNK // NSLOT

    def group(i, carry):
        for s in range(NSLOT):
            c = i * NSLOT + s
            wait_gather(c, s)

            @pl.when(i > 0)
            def _():
                wait_wb(c - NSLOT, s)

            transpose(s)
            fire_wb(c, s)

            @pl.when(i < ngroup - 1)
            def _():
                fire_gather(c + NSLOT, s)

        return carry

    lax.fori_loop(0, ngroup, group, 0)
    for s in range(NSLOT):
        wait_wb(NCHUNK - NSLOT + s, s)


@jax.jit
def _emb(idxt, table):
    mesh = plsc.VectorSubcoreMesh(core_axis_name="c", subcore_axis_name="s")
    k = pl.kernel(
        _emb_body,
        mesh=mesh,
        compiler_params=pltpu.CompilerParams(
            use_tc_tiling_on_sc=False, needs_layout_passes=False
        ),
        out_type=jax.ShapeDtypeStruct((SEQ, NG, BATCH // 128, 8, 128), jnp.float32),
        scratch_types=[
            pltpu.VMEM((SEQ, BW), jnp.int32),
            *[pltpu.VMEM((128, NINP), jnp.float32)] * NSLOT,
            *[pltpu.VMEM((NG, 8, 128), jnp.float32)] * NSLOT,
            [pltpu.SemaphoreType.DMA] * NSLOT,
            [pltpu.SemaphoreType.DMA] * NSLOT,
        ],
    )
    return k(idxt, table)


def kernel(input_ids, embed_weight):
    idxt = input_ids.T.astype(jnp.int32)
    o5 = _emb(idxt, embed_weight)
    return o5.transpose(2, 4, 0, 1, 3).reshape(BATCH, SEQ, NINP)
